# baseline (speedup 1.0000x reference)
import functools

import jax
import jax.numpy as jnp
from jax import lax
from jax.experimental import pallas as pl
from jax.experimental.pallas import tpu as pltpu
from jax.experimental.pallas import tpu_sc as plsc

NUM_EMB = 1000
EMB_DIM = 10
OUT_DIM = 2
BATCH = 16384
HIST = 200

NC = 2
NS = 16
L = 16
NW = NC * NS
ROWS_PER_W = BATCH // NW
GROUPS = ROWS_PER_W // L

U = 8
NACC = 4
CH = 4
NCH = GROUPS // CH
CHW = CH * L * HIST


def _fold_body(table_ref, w_ref, b_ref, pk_ref):
    prod = lax.dot_general(
        w_ref[...], table_ref[...],
        (((1,), (1,)), ((), ())),
        preferred_element_type=jnp.float32,
    )
    t = prod * (1.0 / HIST) + b_ref[...].reshape(OUT_DIM, 1) * (1.0 / HIST)
    bits = lax.bitcast_convert_type(t, jnp.uint32)
    rnd = bits + jnp.uint32(0x7FFF) + ((bits >> 16) & jnp.uint32(1))
    top = rnd & jnp.uint32(0xFFFF0000)
    pk = top[1:2, :] | (top[0:1, :] >> 16)
    pk_ref[...] = lax.bitcast_convert_type(pk, jnp.int32)


def _fold_table(table, W, b):
    return pl.pallas_call(
        _fold_body,
        out_shape=jax.ShapeDtypeStruct((1, NUM_EMB), jnp.int32),
    )(table, W, b)


def _make_sc_kernel():
    mesh = plsc.VectorSubcoreMesh(
        core_axis_name="c", subcore_axis_name="s",
        num_cores=NC, num_subcores=NS,
    )

    @functools.partial(
        pl.kernel,
        out_type=jax.ShapeDtypeStruct((BATCH * OUT_DIM,), jnp.float32),
        mesh=mesh,
        compiler_params=pltpu.CompilerParams(needs_layout_passes=False),
        scratch_types=[
            pltpu.VMEM((NUM_EMB,), jnp.int32),
            pltpu.VMEM((2, CH * L, HIST), jnp.int32),
            pltpu.VMEM((ROWS_PER_W * OUT_DIM,), jnp.float32),
            pltpu.SemaphoreType.DMA,
            pltpu.SemaphoreType.DMA,
        ],
    )
    def sc_embed(x_hbm, t_hbm, out_hbm, t_v, x_v, out_v, sem0, sem1):
        wid = lax.axis_index("s") * NC + lax.axis_index("c")
        row0 = wid * ROWS_PER_W
        pltpu.sync_copy(t_hbm, t_v)

        riota = lax.iota(jnp.int32, L)
        dbase = riota * OUT_DIM
        sems = (sem0, sem1)
        mask_hi = jnp.int32(-65536)

        def chunk_src(c):
            return x_hbm.at[pl.ds(row0 + c * (CH * L), CH * L), :]

        def buf_dst(buf):
            return x_v.at[buf]

        pending = [pltpu.async_copy(chunk_src(0), buf_dst(0), sem0), None]
        for c in range(NCH):
            buf = c & 1
            pending[buf].wait()
            if c + 1 < NCH:
                nb = 1 - buf
                pending[nb] = pltpu.async_copy(
                    chunk_src(c + 1), buf_dst(nb), sems[nb])

            def group_body(g, carry, *, _buf=buf, _c=c):
                lrow = g * L + riota

                def l_body(i, inner):
                    *accs, xoff = inner
                    accs = list(accs)
                    for k in range(U):
                        idx = plsc.load_gather(
                            x_v.at[_buf], [lrow, xoff + k if k else xoff])
                        w = plsc.load_gather(t_v, [idx])
                        v1 = plsc.bitcast(w & mask_hi, jnp.float32)
                        v0 = plsc.bitcast(w << 16, jnp.float32)
                        j = k % NACC
                        accs[2 * j] = accs[2 * j] + v0
                        accs[2 * j + 1] = accs[2 * j + 1] + v1
                    return (*accs, xoff + U)

                z = jnp.zeros((L,), jnp.float32)
                res = lax.fori_loop(
                    0, HIST // U, l_body,
                    ((z,) * (2 * NACC)) + (jnp.zeros((L,), jnp.int32),))
                a0 = (res[0] + res[2]) + (res[4] + res[6])
                a1 = (res[1] + res[3]) + (res[5] + res[7])
                didx = (_c * CH + g) * (L * OUT_DIM) + dbase
                plsc.store_scatter(out_v, [didx], a0)
                plsc.store_scatter(out_v, [didx + 1], a1)
                return carry

            lax.fori_loop(0, CH, group_body, 0)

        pltpu.sync_copy(
            out_v, out_hbm.at[pl.ds(row0 * OUT_DIM, ROWS_PER_W * OUT_DIM)])

    return sc_embed


_sc_embed = _make_sc_kernel()


def kernel(x, table, W, b):
    t = _fold_table(table, W, b).reshape(NUM_EMB)
    out = _sc_embed(x, t)
    return out.reshape(BATCH, OUT_DIM)


# speedup vs baseline: 1.0865x; 1.0865x over previous
import functools

import jax
import jax.numpy as jnp
from jax import lax
from jax.experimental import pallas as pl
from jax.experimental.pallas import tpu as pltpu
from jax.experimental.pallas import tpu_sc as plsc

NUM_EMB = 1000
EMB_DIM = 10
OUT_DIM = 2
BATCH = 16384
HIST = 200

NC = 2
NS = 16
L = 16
NW = NC * NS
ROWS_PER_W = BATCH // NW
GROUPS = ROWS_PER_W // L

U = 8
NACC = 4
CH = 4
NCH = GROUPS // CH
CHW = CH * L * HIST


def _fold_body(table_ref, w_ref, b_ref, pk_ref):
    prod = lax.dot_general(
        w_ref[...], table_ref[...],
        (((1,), (1,)), ((), ())),
        preferred_element_type=jnp.float32,
    )
    t = prod * (1.0 / HIST) + b_ref[...].reshape(OUT_DIM, 1) * (1.0 / HIST)
    bits = lax.bitcast_convert_type(t, jnp.uint32)
    rnd = bits + jnp.uint32(0x7FFF) + ((bits >> 16) & jnp.uint32(1))
    top = rnd & jnp.uint32(0xFFFF0000)
    pk = top[1, :] | (top[0, :] >> 16)
    pk_ref[...] = lax.bitcast_convert_type(pk, jnp.int32)


def _fold_table(table, W, b):
    return pl.pallas_call(
        _fold_body,
        out_shape=jax.ShapeDtypeStruct((NUM_EMB,), jnp.int32),
    )(table, W, b)


def _make_sc_kernel():
    mesh = plsc.VectorSubcoreMesh(
        core_axis_name="c", subcore_axis_name="s",
        num_cores=NC, num_subcores=NS,
    )

    @functools.partial(
        pl.kernel,
        out_type=jax.ShapeDtypeStruct((BATCH, OUT_DIM), jnp.float32),
        mesh=mesh,
        compiler_params=pltpu.CompilerParams(needs_layout_passes=False),
        scratch_types=[
            pltpu.VMEM((NUM_EMB,), jnp.int32),
            pltpu.VMEM((2, CH * L, HIST), jnp.int32),
            pltpu.VMEM((ROWS_PER_W, OUT_DIM), jnp.float32),
            pltpu.SemaphoreType.DMA,
            pltpu.SemaphoreType.DMA,
        ],
    )
    def sc_embed(x_hbm, t_hbm, out_hbm, t_v, x_v, out_v, sem0, sem1):
        wid = lax.axis_index("s") * NC + lax.axis_index("c")
        row0 = wid * ROWS_PER_W
        pltpu.sync_copy(t_hbm, t_v)

        riota = lax.iota(jnp.int32, L)
        zeros = jnp.zeros((L,), jnp.int32)
        ones = zeros + 1
        sems = (sem0, sem1)
        mask_hi = jnp.int32(-65536)

        def chunk_src(c):
            return x_hbm.at[pl.ds(row0 + c * (CH * L), CH * L), :]

        def buf_dst(buf):
            return x_v.at[buf]

        pending = [pltpu.async_copy(chunk_src(0), buf_dst(0), sem0), None]
        for c in range(NCH):
            buf = c & 1
            pending[buf].wait()
            if c + 1 < NCH:
                nb = 1 - buf
                pending[nb] = pltpu.async_copy(
                    chunk_src(c + 1), buf_dst(nb), sems[nb])

            def group_body(g, carry, *, _buf=buf, _c=c):
                lrow = g * L + riota

                def l_body(i, inner):
                    *accs, xoff = inner
                    accs = list(accs)
                    for k in range(U):
                        idx = plsc.load_gather(
                            x_v.at[_buf], [lrow, xoff + k if k else xoff])
                        w = plsc.load_gather(t_v, [idx])
                        v1 = plsc.bitcast(w & mask_hi, jnp.float32)
                        v0 = plsc.bitcast(w << 16, jnp.float32)
                        j = k % NACC
                        accs[2 * j] = accs[2 * j] + v0
                        accs[2 * j + 1] = accs[2 * j + 1] + v1
                    return (*accs, xoff + U)

                z = jnp.zeros((L,), jnp.float32)
                res = lax.fori_loop(
                    0, HIST // U, l_body,
                    ((z,) * (2 * NACC)) + (jnp.zeros((L,), jnp.int32),))
                a0 = (res[0] + res[2]) + (res[4] + res[6])
                a1 = (res[1] + res[3]) + (res[5] + res[7])
                rvec = (_c * CH + g) * L + riota
                plsc.store_scatter(out_v, [rvec, zeros], a0)
                plsc.store_scatter(out_v, [rvec, ones], a1)
                return carry

            lax.fori_loop(0, CH, group_body, 0)

        pltpu.sync_copy(out_v, out_hbm.at[pl.ds(row0, ROWS_PER_W), :])

    return sc_embed


_sc_embed = _make_sc_kernel()


def kernel(x, table, W, b):
    t = _fold_table(table, W, b)
    return _sc_embed(x, t)


# speedup vs baseline: 3.0666x; 2.8224x over previous
import functools

import jax
import jax.numpy as jnp
from jax import lax
from jax.experimental import pallas as pl
from jax.experimental.pallas import tpu as pltpu
from jax.experimental.pallas import tpu_sc as plsc

NUM_EMB = 1000
EMB_DIM = 10
OUT_DIM = 2
BATCH = 16384
HIST = 200

NC = 2
NS = 16
L = 16
NW = NC * NS
COLS_PER_W = BATCH // NW
GROUPS = COLS_PER_W // L

U = 8
NACC = 4
CH = 8
NCH = GROUPS // CH
CHC = CH * L


def _fold_body(table_ref, w_ref, b_ref, pk_ref):
    prod = lax.dot_general(
        w_ref[...], table_ref[...],
        (((1,), (1,)), ((), ())),
        preferred_element_type=jnp.float32,
    )
    t = prod * (1.0 / HIST) + b_ref[...].reshape(OUT_DIM, 1) * (1.0 / HIST)
    bits = lax.bitcast_convert_type(t, jnp.uint32)
    rnd = bits + jnp.uint32(0x7FFF) + ((bits >> 16) & jnp.uint32(1))
    top = rnd & jnp.uint32(0xFFFF0000)
    pk = top[1, :] | (top[0, :] >> 16)
    pk_ref[...] = lax.bitcast_convert_type(pk, jnp.int32)


def _fold_table(table, W, b):
    return pl.pallas_call(
        _fold_body,
        out_shape=jax.ShapeDtypeStruct((NUM_EMB,), jnp.int32),
    )(table, W, b)


def _make_sc_kernel():
    mesh = plsc.VectorSubcoreMesh(
        core_axis_name="c", subcore_axis_name="s",
        num_cores=NC, num_subcores=NS,
    )

    @functools.partial(
        pl.kernel,
        out_type=jax.ShapeDtypeStruct((BATCH * OUT_DIM,), jnp.float32),
        mesh=mesh,
        compiler_params=pltpu.CompilerParams(needs_layout_passes=False),
        scratch_types=[
            pltpu.VMEM((NUM_EMB,), jnp.int32),
            pltpu.VMEM((2, HIST, CHC), jnp.int32),
            pltpu.VMEM((COLS_PER_W * OUT_DIM,), jnp.float32),
            pltpu.SemaphoreType.DMA,
            pltpu.SemaphoreType.DMA,
        ],
    )
    def sc_embed(xt_hbm, t_hbm, out_hbm, t_v, x_v, out_v, sem0, sem1):
        wid = lax.axis_index("s") * NC + lax.axis_index("c")
        col0 = wid * COLS_PER_W
        pltpu.sync_copy(t_hbm, t_v)

        sems = (sem0, sem1)
        mask_hi = jnp.int32(-65536)

        def chunk_src(c):
            return xt_hbm.at[:, pl.ds(col0 + c * CHC, CHC)]

        def buf_dst(buf):
            return x_v.at[buf]

        pending = [pltpu.async_copy(chunk_src(0), buf_dst(0), sem0), None]
        for c in range(NCH):
            buf = c & 1
            pending[buf].wait()
            if c + 1 < NCH:
                nb = 1 - buf
                pending[nb] = pltpu.async_copy(
                    chunk_src(c + 1), buf_dst(nb), sems[nb])

            def group_body(g, carry, *, _buf=buf, _c=c):
                cb = g * L

                def l_body(i, accs):
                    accs = list(accs)
                    lb = i * U
                    for k in range(U):
                        idx = x_v[_buf, lb + k, pl.ds(cb, L)]
                        w = plsc.load_gather(t_v, [idx])
                        v1 = plsc.bitcast(w & mask_hi, jnp.float32)
                        v0 = plsc.bitcast(w << 16, jnp.float32)
                        j = k % NACC
                        accs[2 * j] = accs[2 * j] + v0
                        accs[2 * j + 1] = accs[2 * j + 1] + v1
                    return tuple(accs)

                z = jnp.zeros((L,), jnp.float32)
                res = lax.fori_loop(0, HIST // U, l_body, (z,) * (2 * NACC))
                a0 = (res[0] + res[2]) + (res[4] + res[6])
                a1 = (res[1] + res[3]) + (res[5] + res[7])
                g2 = _c * CH + g
                doff = (g2 // 8) * 256 + (g2 % 8) * L
                out_v[pl.ds(doff, L)] = a0
                out_v[pl.ds(doff + 128, L)] = a1
                return carry

            lax.fori_loop(0, CH, group_body, 0)

        pltpu.sync_copy(
            out_v,
            out_hbm.at[pl.ds(wid * (COLS_PER_W * OUT_DIM),
                             COLS_PER_W * OUT_DIM)])

    return sc_embed


_sc_embed = _make_sc_kernel()


def kernel(x, table, W, b):
    t = _fold_table(table, W, b)
    flat = _sc_embed(x.T, t)
    return (flat.reshape(BATCH // 128, OUT_DIM, 128)
            .transpose(0, 2, 1)
            .reshape(BATCH, OUT_DIM))
